# Initial kernel scaffold; baseline (speedup 1.0000x reference)
import functools

import jax
import jax.numpy as jnp
from jax import lax
from jax.experimental import pallas as pl
from jax.experimental.pallas import tpu as pltpu
from jax.experimental.pallas import tpu_sc as plsc

N = 10000
E = 320000
IN_DIM = 128
HID_DIM = 256
OUT_DIM = 128
K = 10

T = 16
C = 80
CH = E // (T * C)
NB = N // T
R = 125
M = NB // R
D = OUT_DIM


def _mlp_body(x_ref, w1_ref, b1_ref, w2_ref, b2_ref, o_ref):
    h = jnp.maximum(
        jnp.dot(x_ref[...], w1_ref[...], preferred_element_type=jnp.float32)
        + b1_ref[...][None, :], 0.0)
    o_ref[...] = (
        jnp.dot(h, w2_ref[...], preferred_element_type=jnp.float32)
        + b2_ref[...][None, :])


def _mlp(x, W1, b1, W2, b2):
    BR = 1000
    return pl.pallas_call(
        _mlp_body,
        grid=(N // BR,),
        in_specs=[
            pl.BlockSpec((BR, IN_DIM), lambda i: (i, 0)),
            pl.BlockSpec((IN_DIM, HID_DIM), lambda i: (0, 0)),
            pl.BlockSpec((HID_DIM,), lambda i: (0,)),
            pl.BlockSpec((HID_DIM, OUT_DIM), lambda i: (0, 0)),
            pl.BlockSpec((OUT_DIM,), lambda i: (0,)),
        ],
        out_specs=pl.BlockSpec((BR, OUT_DIM), lambda i: (i, 0)),
        out_shape=jax.ShapeDtypeStruct((N, OUT_DIM), jnp.float32),
    )(x, W1, b1, W2, b2)


def _rsqrt16(v):
    i = plsc.bitcast(v, jnp.int32)
    i = jnp.int32(0x5F3759DF) - lax.shift_right_arithmetic(i, jnp.int32(1))
    y = plsc.bitcast(i, jnp.float32)
    vh = v * jnp.float32(-0.5)
    for _ in range(4):
        y = y * (jnp.float32(1.5) + vh * y * y)
    return jnp.where(v > 0.0, y, jnp.float32(0.0))


def _sc_body(h0_hbm, src_hbm, dst_hbm,
             hout_hbm, g_hbm,
             degs_sh, degd_sh, nd_sh, nc_sh, agg_sh,
             idx_v, idx2_v, ones_v, rows_v, zero16_v, zerod_v,
             da_v, db_v, na_v, nb_v, nc_v, acc_v, hbuf_v, gbuf_v,
             sem):
    tid = lax.axis_index("s")

    def zdeg(m, _):
        base = tid * NB + m * R
        pltpu.sync_copy(zero16_v, degs_sh.at[pl.ds(base, R)])
        pltpu.sync_copy(zero16_v, degd_sh.at[pl.ds(base, R)])
        return 0
    lax.fori_loop(0, M, zdeg, 0)
    plsc.subcore_barrier()

    def deg_body(j, _):
        pltpu.sync_copy(src_hbm.at[tid, j], idx_v)
        pltpu.sync_copy(dst_hbm.at[tid, j], idx2_v)
        pltpu.sync_copy(ones_v, degs_sh.at[idx_v], add=True)
        pltpu.sync_copy(ones_v, degd_sh.at[idx2_v], add=True)
        return 0
    lax.fori_loop(0, CH, deg_body, 0)
    plsc.subcore_barrier()

    def norm_body(m, _):
        base = tid * NB + m * R
        pltpu.sync_copy(degs_sh.at[pl.ds(base, R)], da_v)
        pltpu.sync_copy(degd_sh.at[pl.ds(base, R)], db_v)

        def nrow(r, _):
            ns = _rsqrt16(da_v[r])
            ndv = _rsqrt16(db_v[r])
            na_v[r] = ns
            nb_v[r] = ndv
            nc_v[r] = ns * ndv
            return 0
        lax.fori_loop(0, R, nrow, 0)
        pltpu.sync_copy(nb_v, nd_sh.at[pl.ds(base, R)])
        pltpu.sync_copy(nc_v, nc_sh.at[pl.ds(base, R)])

        pltpu.sync_copy(h0_hbm.at[pl.ds(base, R)], hbuf_v)

        def grow(r, _):
            s = na_v[r, 0]
            for c in range(D // 16):
                gbuf_v[r, pl.ds(c * 16, 16)] = hbuf_v[r, pl.ds(c * 16, 16)] * s
            return 0
        lax.fori_loop(0, R, grow, 0)
        pltpu.sync_copy(gbuf_v, g_hbm.at[pl.ds(base, R)])
        pltpu.sync_copy(zerod_v, agg_sh.at[pl.ds(base, R)])
        return 0
    lax.fori_loop(0, M, norm_body, 0)
    plsc.subcore_barrier()

    for k in range(K):
        def edge_body(j, _):
            pltpu.sync_copy(src_hbm.at[tid, j], idx_v)
            pltpu.sync_copy(dst_hbm.at[tid, j], idx2_v)
            pltpu.async_copy(g_hbm.at[idx_v], rows_v, sem).wait()
            pltpu.sync_copy(rows_v, agg_sh.at[idx2_v], add=True)
            return 0
        lax.fori_loop(0, CH, edge_body, 0)
        plsc.subcore_barrier()

        def wb_body(m, _):
            base = tid * NB + m * R
            pltpu.sync_copy(agg_sh.at[pl.ds(base, R)], acc_v)
            pltpu.sync_copy(nd_sh.at[pl.ds(base, R)], nb_v)
            pltpu.sync_copy(nc_sh.at[pl.ds(base, R)], nc_v)

            def srow(r, _):
                ndv = nb_v[r, 0]
                ncv = nc_v[r, 0]
                for c in range(D // 16):
                    v = acc_v[r, pl.ds(c * 16, 16)]
                    hbuf_v[r, pl.ds(c * 16, 16)] = v * ndv
                    gbuf_v[r, pl.ds(c * 16, 16)] = v * ncv
                return 0
            lax.fori_loop(0, R, srow, 0)

            pltpu.sync_copy(hbuf_v, hout_hbm.at[k, pl.ds(base, R)])
            if k < K - 1:
                pltpu.sync_copy(gbuf_v, g_hbm.at[pl.ds(base, R)])
                pltpu.sync_copy(zerod_v, agg_sh.at[pl.ds(base, R)])
            return 0
        lax.fori_loop(0, M, wb_body, 0)
        plsc.subcore_barrier()


def _sc_hops(h0, src3, dst3):
    mesh = plsc.VectorSubcoreMesh(
        core_axis_name="c", subcore_axis_name="s", num_cores=1)
    f = functools.partial(
        pl.kernel,
        out_type=[
            jax.ShapeDtypeStruct((K, N, D), jnp.float32),
            jax.ShapeDtypeStruct((N, D), jnp.float32),
        ],
        mesh=mesh,
        scratch_types=[
            pltpu.VMEM_SHARED((N, 16), jnp.float32),
            pltpu.VMEM_SHARED((N, 16), jnp.float32),
            pltpu.VMEM_SHARED((N, 16), jnp.float32),
            pltpu.VMEM_SHARED((N, 16), jnp.float32),
            pltpu.VMEM_SHARED((N, D), jnp.float32),
            pltpu.VMEM((C,), jnp.int32),
            pltpu.VMEM((C,), jnp.int32),
            pltpu.VMEM((C, 16), jnp.float32),
            pltpu.VMEM((C, D), jnp.float32),
            pltpu.VMEM((R, 16), jnp.float32),
            pltpu.VMEM((R, D), jnp.float32),
            pltpu.VMEM((R, 16), jnp.float32),
            pltpu.VMEM((R, 16), jnp.float32),
            pltpu.VMEM((R, 16), jnp.float32),
            pltpu.VMEM((R, 16), jnp.float32),
            pltpu.VMEM((R, 16), jnp.float32),
            pltpu.VMEM((R, D), jnp.float32),
            pltpu.VMEM((R, D), jnp.float32),
            pltpu.VMEM((R, D), jnp.float32),
            pltpu.SemaphoreType.DMA,
        ],
    )
    return f(_sc_body)(h0, src3, dst3)


def _gate_body(h0_ref, hh_ref, pw_ref, pb_ref, o_ref):
    pw = pw_ref[...]
    pb = pb_ref[...]
    h0 = h0_ref[...]
    s = jax.nn.sigmoid(jnp.dot(h0, pw, preferred_element_type=jnp.float32) + pb)
    acc = s * h0
    for k in range(K):
        hk = hh_ref[k]
        sk = jax.nn.sigmoid(
            jnp.dot(hk, pw, preferred_element_type=jnp.float32) + pb)
        acc = acc + sk * hk
    o_ref[...] = acc


def _gating(h0, hh, proj_w, proj_b):
    BR = 500
    return pl.pallas_call(
        _gate_body,
        grid=(N // BR,),
        in_specs=[
            pl.BlockSpec((BR, D), lambda i: (i, 0)),
            pl.BlockSpec((K, BR, D), lambda i: (0, i, 0)),
            pl.BlockSpec((D, 1), lambda i: (0, 0)),
            pl.BlockSpec((1,), lambda i: (0,)),
        ],
        out_specs=pl.BlockSpec((BR, D), lambda i: (i, 0)),
        out_shape=jax.ShapeDtypeStruct((N, D), jnp.float32),
    )(h0, hh, proj_w, proj_b)


@jax.jit
def kernel(x, edge_index, W1, b1, W2, b2, proj_w, proj_b):
    h0 = _mlp(x, W1, b1, W2, b2)
    src3 = edge_index[0].reshape(T, CH, C)
    dst3 = edge_index[1].reshape(T, CH, C)
    hh, _g = _sc_hops(h0, src3, dst3)
    return _gating(h0, hh, proj_w, proj_b)


# speedup vs baseline: 2.5675x; 2.5675x over previous
import functools

import jax
import jax.numpy as jnp
from jax import lax
from jax.experimental import pallas as pl
from jax.experimental.pallas import tpu as pltpu
from jax.experimental.pallas import tpu_sc as plsc

N = 10000
E = 320000
IN_DIM = 128
HID_DIM = 256
OUT_DIM = 128
K = 10

T = 16
C = 80
CH = E // (T * C)
RB = 40
NCHN = N // RB
NIT = -(-NCHN // T)
D = OUT_DIM


def _mlp_body(x_ref, w1_ref, b1_ref, w2_ref, b2_ref, o_ref):
    h = jnp.maximum(
        jnp.dot(x_ref[...], w1_ref[...], preferred_element_type=jnp.float32)
        + b1_ref[...][None, :], 0.0)
    o_ref[...] = (
        jnp.dot(h, w2_ref[...], preferred_element_type=jnp.float32)
        + b2_ref[...][None, :])


def _mlp(x, W1, b1, W2, b2):
    BR = 1000
    return pl.pallas_call(
        _mlp_body,
        grid=(N // BR,),
        in_specs=[
            pl.BlockSpec((BR, IN_DIM), lambda i: (i, 0)),
            pl.BlockSpec((IN_DIM, HID_DIM), lambda i: (0, 0)),
            pl.BlockSpec((HID_DIM,), lambda i: (0,)),
            pl.BlockSpec((HID_DIM, OUT_DIM), lambda i: (0, 0)),
            pl.BlockSpec((OUT_DIM,), lambda i: (0,)),
        ],
        out_specs=pl.BlockSpec((BR, OUT_DIM), lambda i: (i, 0)),
        out_shape=jax.ShapeDtypeStruct((N, OUT_DIM), jnp.float32),
    )(x, W1, b1, W2, b2)


def _rsqrt16(v):
    i = lax.bitcast_convert_type(v, jnp.int32)
    i = jnp.int32(0x5F3759DF) - lax.shift_right_arithmetic(i, jnp.int32(1))
    y = lax.bitcast_convert_type(i, jnp.float32)
    vh = v * jnp.float32(-0.5)
    for _ in range(4):
        y = y * (jnp.float32(1.5) + vh * y * y)
    return jnp.where(v > 0.0, y, jnp.float32(0.0))


def _mesh():
    return plsc.VectorSubcoreMesh(
        core_axis_name="c", subcore_axis_name="s", num_cores=1)


def _node_loop(tid, body):
    def outer(i, _):
        cid = i * T + tid

        @pl.when(cid < NCHN)
        def _():
            body(pl.multiple_of(cid * RB, RB))
        return 0
    lax.fori_loop(0, NIT, outer, 0)


def _edge_off(tid, j):
    return pl.multiple_of((tid * CH + j) * C, C)


def _degnorm_body(src_hbm, dst_hbm,
                  ns_hbm, nd_hbm,
                  hist_sh,
                  idx_v, ones_v, zerod_v, acc_v):
    tid = lax.axis_index("s")

    zeros16 = jnp.zeros((16,), jnp.float32)
    ones16 = jnp.ones((16,), jnp.float32)

    def initz_row(r, _):
        for c in range(D // 16):
            zerod_v[r, pl.ds(c * 16, 16)] = zeros16
        return 0
    lax.fori_loop(0, RB, initz_row, 0)

    def init1_row(r, _):
        for c in range(D // 16):
            ones_v[r, pl.ds(c * 16, 16)] = ones16
        return 0
    lax.fori_loop(0, C, init1_row, 0)

    def zero_hist(base):
        pltpu.sync_copy(zerod_v, hist_sh.at[pl.ds(base, RB)])
    _node_loop(tid, zero_hist)
    plsc.subcore_barrier()

    def round_(edges_hbm, out_hbm, last):
        def deg_body(j, _):
            off = _edge_off(tid, j)
            pltpu.sync_copy(edges_hbm.at[pl.ds(off, C)], idx_v)
            pltpu.sync_copy(ones_v, hist_sh.at[idx_v], add=True)
            return 0
        lax.fori_loop(0, CH, deg_body, 0)
        plsc.subcore_barrier()

        def norm_body(base):
            pltpu.sync_copy(hist_sh.at[pl.ds(base, RB)], acc_v)
            if not last:
                pltpu.sync_copy(zerod_v, hist_sh.at[pl.ds(base, RB)])

            def nrow(r, _):
                for c in range(D // 16):
                    sl = pl.ds(c * 16, 16)
                    acc_v[r, sl] = _rsqrt16(acc_v[r, sl])
                return 0
            lax.fori_loop(0, RB, nrow, 0)
            pltpu.sync_copy(acc_v, out_hbm.at[pl.ds(base, RB)])
        _node_loop(tid, norm_body)
        plsc.subcore_barrier()

    round_(src_hbm, ns_hbm, False)
    round_(dst_hbm, nd_hbm, True)


def _sc_degnorm(src, dst):
    f = functools.partial(
        pl.kernel,
        out_type=[
            jax.ShapeDtypeStruct((N, D), jnp.float32),
            jax.ShapeDtypeStruct((N, D), jnp.float32),
        ],
        mesh=_mesh(),
        scratch_types=[
            pltpu.VMEM_SHARED((N, D), jnp.float32),
            pltpu.VMEM((C,), jnp.int32),
            pltpu.VMEM((C, D), jnp.float32),
            pltpu.VMEM((RB, D), jnp.float32),
            pltpu.VMEM((RB, D), jnp.float32),
        ],
    )
    return f(_degnorm_body)(src, dst)


def _hops_body(h0_hbm, src_hbm, dst_hbm, ns_hbm, nd_hbm,
               hout_hbm, g_hbm,
               agg_sh,
               idx_v, idx2_v, rows_v, zerod_v, ns_b, nd_b, acc_v,
               sem):
    tid = lax.axis_index("s")

    zeros16 = jnp.zeros((16,), jnp.float32)

    def initz_row(r, _):
        for c in range(D // 16):
            zerod_v[r, pl.ds(c * 16, 16)] = zeros16
        return 0
    lax.fori_loop(0, RB, initz_row, 0)

    def scale_rows(dst_buf, src_buf, n_buf):
        def srow(r, _):
            for c in range(D // 16):
                sl = pl.ds(c * 16, 16)
                dst_buf[r, sl] = src_buf[r, sl] * n_buf[r, sl]
            return 0
        lax.fori_loop(0, RB, srow, 0)

    def g0_body(base):
        pltpu.sync_copy(ns_hbm.at[pl.ds(base, RB)], ns_b)
        pltpu.sync_copy(h0_hbm.at[pl.ds(base, RB)], acc_v)
        scale_rows(acc_v, acc_v, ns_b)
        pltpu.sync_copy(acc_v, g_hbm.at[pl.ds(base, RB)])
        pltpu.sync_copy(zerod_v, agg_sh.at[pl.ds(base, RB)])
    _node_loop(tid, g0_body)
    plsc.subcore_barrier()

    def hop(k, _):
        def edge_body(j, _):
            off = _edge_off(tid, j)
            pltpu.sync_copy(src_hbm.at[pl.ds(off, C)], idx_v)
            pltpu.sync_copy(dst_hbm.at[pl.ds(off, C)], idx2_v)
            pltpu.async_copy(g_hbm.at[idx_v], rows_v, sem).wait()
            pltpu.sync_copy(rows_v, agg_sh.at[idx2_v], add=True)
            return 0
        lax.fori_loop(0, CH, edge_body, 0)
        plsc.subcore_barrier()

        def wb_body(base):
            pltpu.sync_copy(agg_sh.at[pl.ds(base, RB)], acc_v)
            pltpu.sync_copy(nd_hbm.at[pl.ds(base, RB)], nd_b)
            scale_rows(acc_v, acc_v, nd_b)
            hrow = pl.multiple_of(k * N + base, 8)
            pltpu.sync_copy(acc_v, hout_hbm.at[pl.ds(hrow, RB)])

            @pl.when(k < K - 1)
            def _():
                pltpu.sync_copy(ns_hbm.at[pl.ds(base, RB)], ns_b)
                scale_rows(acc_v, acc_v, ns_b)
                pltpu.sync_copy(acc_v, g_hbm.at[pl.ds(base, RB)])
                pltpu.sync_copy(zerod_v, agg_sh.at[pl.ds(base, RB)])
        _node_loop(tid, wb_body)
        plsc.subcore_barrier()
        return 0
    lax.fori_loop(0, K, hop, 0)


def _sc_hops(h0, src, dst, ns, nd):
    f = functools.partial(
        pl.kernel,
        out_type=[
            jax.ShapeDtypeStruct((K * N, D), jnp.float32),
            jax.ShapeDtypeStruct((N, D), jnp.float32),
        ],
        mesh=_mesh(),
        scratch_types=[
            pltpu.VMEM_SHARED((N, D), jnp.float32),
            pltpu.VMEM((C,), jnp.int32),
            pltpu.VMEM((C,), jnp.int32),
            pltpu.VMEM((C, D), jnp.float32),
            pltpu.VMEM((RB, D), jnp.float32),
            pltpu.VMEM((RB, D), jnp.float32),
            pltpu.VMEM((RB, D), jnp.float32),
            pltpu.VMEM((RB, D), jnp.float32),
            pltpu.SemaphoreType.DMA,
        ],
    )
    return f(_hops_body)(h0, src, dst, ns, nd)


def _gate_body(h0_ref, hh_ref, pw_ref, pb_ref, o_ref):
    pw = pw_ref[...]
    pb = pb_ref[...]
    h0 = h0_ref[...]
    s = jax.nn.sigmoid(jnp.dot(h0, pw, preferred_element_type=jnp.float32) + pb)
    acc = s * h0
    for k in range(K):
        hk = hh_ref[k]
        sk = jax.nn.sigmoid(
            jnp.dot(hk, pw, preferred_element_type=jnp.float32) + pb)
        acc = acc + sk * hk
    o_ref[...] = acc


def _gating(h0, hh, proj_w, proj_b):
    BR = 1000
    return pl.pallas_call(
        _gate_body,
        grid=(N // BR,),
        in_specs=[
            pl.BlockSpec((BR, D), lambda i: (i, 0)),
            pl.BlockSpec((K, BR, D), lambda i: (0, i, 0)),
            pl.BlockSpec((D, 1), lambda i: (0, 0)),
            pl.BlockSpec((1,), lambda i: (0,)),
        ],
        out_specs=pl.BlockSpec((BR, D), lambda i: (i, 0)),
        out_shape=jax.ShapeDtypeStruct((N, D), jnp.float32),
    )(h0, hh, proj_w, proj_b)


@jax.jit
def kernel(x, edge_index, W1, b1, W2, b2, proj_w, proj_b):
    h0 = _mlp(x, W1, b1, W2, b2)
    src, dst = edge_index[0], edge_index[1]
    ns, nd = _sc_degnorm(src, dst)
    hh, _g = _sc_hops(h0, src, dst, ns, nd)
    return _gating(h0, hh.reshape(K, N, D), proj_w, proj_b)
